# baseline (speedup 1.0000x reference)
import functools

import jax
import jax.numpy as jnp
from jax import lax
from jax.experimental import pallas as pl
from jax.experimental.pallas import tpu as pltpu
from jax.experimental.pallas import tpu_sc as plsc

N = 10000
E = 320000
D = 128
H = 128

NC = 2
NS = 16
NW = NC * NS

NPAD = 10240
RPT = NPAD // NS
EW = E // NW
EC = 80
NCH = EW // EC

@functools.partial(
    pl.kernel,
    out_type=[
        jax.ShapeDtypeStruct((NC, NPAD, D), jnp.float32),
        jax.ShapeDtypeStruct((NC * NPAD * 16,), jnp.float32),
    ],
    mesh=plsc.VectorSubcoreMesh(
        core_axis_name="c", subcore_axis_name="s", num_cores=NC,
        num_subcores=NS),
    scratch_types=[
        [pltpu.VMEM((EC,), jnp.int32)] * 4,
        [pltpu.VMEM((EC,), jnp.int32)] * 4,
        [pltpu.VMEM((EC, D), jnp.float32)] * 4,
        pltpu.VMEM((EC,), jnp.float32),
        pltpu.VMEM((RPT,), jnp.float32),
        [pltpu.VMEM((EC * 16,), jnp.float32)] * 2,
        pltpu.VMEM_SHARED((NPAD, D), jnp.float32),
        pltpu.VMEM_SHARED((NPAD,), jnp.float32),
        [pltpu.SemaphoreType.DMA] * 4,
        [pltpu.SemaphoreType.DMA] * 4,
        [pltpu.SemaphoreType.DMA] * 4,
        [pltpu.SemaphoreType.DMA] * 4,
        pltpu.SemaphoreType.DMA,
    ],
)
def _sc_accumulate(x_hbm, src_hbm, dst_hbm, psum_hbm, pdeg_hbm,
                   sidx, didx, rowsb, ones, dzero, dstage, acc, dacc,
                   gsem, ssem, isem, jsem, dsem):
  cid = lax.axis_index("c")
  sid = lax.axis_index("s")
  wid = cid * NS + sid
  base = wid * EW
  rows0 = rowsb[0]

  def _zrow(r, _):
    for c in range(D // 16):
      rows0[r, pl.ds(c * 16, 16)] = jnp.zeros((16,), jnp.float32)
    return _
  lax.fori_loop(0, EC, _zrow, None)
  for k in range(RPT // EC):
    pltpu.async_copy(rows0, acc.at[pl.ds(sid * RPT + k * EC, EC)], ssem[0])

  def _zdeg(i, _):
    dzero[pl.ds(i * 16, 16)] = jnp.zeros((16,), jnp.float32)
    return _
  lax.fori_loop(0, RPT // 16, _zdeg, None)
  pltpu.async_copy(dzero, dacc.at[pl.ds(sid * RPT, RPT)], ssem[1])

  for i in range(EC // 16):
    ones[pl.ds(i * 16, 16)] = jnp.ones((16,), jnp.float32)

  for k in range(RPT // EC):
    pltpu.make_async_copy(rows0, acc.at[pl.ds(sid * RPT, EC)], ssem[0]).wait()
  pltpu.make_async_copy(dzero, dacc.at[pl.ds(sid * RPT, RPT)], ssem[1]).wait()

  plsc.subcore_barrier()

  for k in range(3):
    pltpu.sync_copy(src_hbm.at[pl.ds(base + k * EC, EC)], sidx[k])
    pltpu.sync_copy(dst_hbm.at[pl.ds(base + k * EC, EC)], didx[k])
    pltpu.async_copy(x_hbm.at[sidx[k]], rowsb[k], gsem[k])
  pltpu.async_copy(src_hbm.at[pl.ds(base + 3 * EC, EC)], sidx[3], isem[3])

  def _step(i, b):
    p = (b + 3) % 4
    pltpu.make_async_copy(x_hbm.at[sidx[b]], rowsb[b], gsem[b]).wait()

    @pl.when(i >= 3)
    def _():
      pltpu.make_async_copy(dst_hbm.at[pl.ds(base, EC)], didx[b],
                            jsem[b]).wait()

    pltpu.async_copy(rowsb[b], acc.at[didx[b]], ssem[b], add=True)
    pltpu.async_copy(ones, dacc.at[didx[b]], dsem, add=True)

    @pl.when(i > 0)
    def _():
      pltpu.make_async_copy(rowsb[p], acc.at[didx[p]], ssem[p]).wait()
      pltpu.make_async_copy(ones, dacc.at[didx[p]], dsem).wait()

    @pl.when(i + 3 < NCH)
    def _():
      pltpu.make_async_copy(src_hbm.at[pl.ds(base, EC)], sidx[p],
                            isem[p]).wait()
      pltpu.async_copy(x_hbm.at[sidx[p]], rowsb[p], gsem[p])
      pltpu.async_copy(dst_hbm.at[pl.ds(base + (i + 3) * EC, EC)],
                       didx[p], jsem[p])

    @pl.when(i + 4 < NCH)
    def _():
      pltpu.async_copy(src_hbm.at[pl.ds(base + (i + 4) * EC, EC)],
                       sidx[b], isem[b])

  def _quad(g, _):
    for b in range(4):
      _step(4 * g + b, b)
    return _
  lax.fori_loop(0, NCH // 4, _quad, None)
  _step(NCH - 1, 0)

  pltpu.make_async_copy(rowsb[0], acc.at[didx[0]], ssem[0]).wait()
  pltpu.make_async_copy(ones, dacc.at[didx[0]], dsem).wait()

  plsc.subcore_barrier()

  sl = pl.ds(sid * RPT, RPT)
  pltpu.async_copy(acc.at[sl], psum_hbm.at[cid, sl], gsem[0])
  pltpu.sync_copy(dacc.at[sl], dzero)

  dbase = (cid * NPAD + sid * RPT) * 16

  for k in range(RPT // EC):
    buf = dstage[k % 2]
    xsem = gsem[1 + (k % 2)]
    if k >= 2:
      pltpu.make_async_copy(
          buf, pdeg_hbm.at[pl.ds(dbase + (k - 2) * EC * 16, EC * 16)],
          xsem).wait()

    def _bg(g, _):
      dv = dzero[pl.ds(k * EC + g * 16, 16)]
      for j in range(16):
        buf[pl.ds((g * 16 + j) * 16, 16)] = jnp.full((16,), dv[j],
                                                     jnp.float32)
      return _
    lax.fori_loop(0, EC // 16, _bg, None)
    pltpu.async_copy(buf, pdeg_hbm.at[pl.ds(dbase + k * EC * 16, EC * 16)],
                     xsem)

  for k in range(RPT // EC - 2, RPT // EC):
    pltpu.make_async_copy(
        dstage[k % 2],
        pdeg_hbm.at[pl.ds(dbase + k * EC * 16, EC * 16)],
        gsem[1 + (k % 2)]).wait()
  pltpu.make_async_copy(acc.at[sl], psum_hbm.at[cid, sl], gsem[0]).wait()


BN = 2000


def _mlp_body(x_ref, ps_ref, pd_ref, w1_ref, b1_ref, w2_ref, b2_ref, w3_ref,
              b3_ref, o_ref):
  deg = pd_ref[0][:, 0:1] + pd_ref[1][:, 0:1]
  has = deg > 0.0
  sn = jnp.where(has, 0.5 / jnp.maximum(deg, 1.0), 0.0)
  sx = jnp.where(has, 0.5, 1.0)
  a = x_ref[...] * sx + (ps_ref[0] + ps_ref[1]) * sn
  dn = (((1,), (1,)), ((), ()))
  bf = jnp.bfloat16
  h = lax.dot_general(a.astype(bf), w1_ref[...].astype(bf), dn,
                      preferred_element_type=jnp.float32)
  h = jnp.maximum(h + b1_ref[...], 0.0)
  h = lax.dot_general(h.astype(bf), w2_ref[...].astype(bf), dn,
                      preferred_element_type=jnp.float32)
  h = jnp.maximum(h + b2_ref[...], 0.0)
  o_ref[...] = jnp.sum(h * w3_ref[...], axis=1, keepdims=True) + b3_ref[...]


def _tc_mlp(x, psum, pdegx, W1, b1, W2, b2, W3, b3):
  return pl.pallas_call(
      _mlp_body,
      grid=(N // BN,),
      in_specs=[
          pl.BlockSpec((BN, D), lambda g: (g, 0)),
          pl.BlockSpec((NC, BN, D), lambda g: (0, g, 0)),
          pl.BlockSpec((NC, BN, 16), lambda g: (0, g, 0)),
          pl.BlockSpec((H, D), lambda g: (0, 0)),
          pl.BlockSpec((1, H), lambda g: (0, 0)),
          pl.BlockSpec((H, H), lambda g: (0, 0)),
          pl.BlockSpec((1, H), lambda g: (0, 0)),
          pl.BlockSpec((1, H), lambda g: (0, 0)),
          pl.BlockSpec((1, 1), lambda g: (0, 0)),
      ],
      out_specs=pl.BlockSpec((BN, 1), lambda g: (g, 0)),
      out_shape=jax.ShapeDtypeStruct((N, 1), jnp.float32),
  )(x, psum, pdegx, W1, b1.reshape(1, H), W2, b2.reshape(1, H), W3,
    b3.reshape(1, 1))


def kernel(x, edge_index, W1, b1, W2, b2, W3, b3):
  src = edge_index[0]
  dst = edge_index[1]
  psum, pdeg_flat = _sc_accumulate(x, src, dst)
  pdegx = pdeg_flat.reshape(NC, NPAD, 16)
  return _tc_mlp(x, psum, pdegx, W1, b1, W2, b2, W3, b3)


# speedup vs baseline: 1.0249x; 1.0249x over previous
import functools

import jax
import jax.numpy as jnp
from jax import lax
from jax.experimental import pallas as pl
from jax.experimental.pallas import tpu as pltpu
from jax.experimental.pallas import tpu_sc as plsc

N = 10000
E = 320000
D = 128
H = 128

NC = 2
NS = 16
NW = NC * NS

NPAD = 10240
RPT = NPAD // NS
EW = E // NW
EC = 80
NCH = EW // EC

@functools.partial(
    pl.kernel,
    out_type=[
        jax.ShapeDtypeStruct((NC, NPAD, D), jnp.float32),
        jax.ShapeDtypeStruct((NC, NPAD, D), jnp.float32),
    ],
    mesh=plsc.VectorSubcoreMesh(
        core_axis_name="c", subcore_axis_name="s", num_cores=NC,
        num_subcores=NS),
    scratch_types=[
        [pltpu.VMEM((EC,), jnp.int32)] * 4,
        [pltpu.VMEM((EC,), jnp.int32)] * 4,
        [pltpu.VMEM((EC, D), jnp.float32)] * 4,
        pltpu.VMEM((EC,), jnp.float32),
        pltpu.VMEM((RPT,), jnp.float32),
        pltpu.VMEM_SHARED((NPAD, D), jnp.float32),
        pltpu.VMEM_SHARED((NPAD,), jnp.float32),
        [pltpu.SemaphoreType.DMA] * 4,
        [pltpu.SemaphoreType.DMA] * 4,
        [pltpu.SemaphoreType.DMA] * 4,
        [pltpu.SemaphoreType.DMA] * 4,
        pltpu.SemaphoreType.DMA,
    ],
)
def _sc_accumulate(x_hbm, src_hbm, dst_hbm, psum_hbm, pdeg_hbm,
                   sidx, didx, rowsb, ones, dzero, acc, dacc,
                   gsem, ssem, isem, jsem, dsem):
  cid = lax.axis_index("c")
  sid = lax.axis_index("s")
  wid = cid * NS + sid
  base = wid * EW
  rows0 = rowsb[0]

  def _zrow(r, _):
    for c in range(D // 16):
      rows0[r, pl.ds(c * 16, 16)] = jnp.zeros((16,), jnp.float32)
    return _
  lax.fori_loop(0, EC, _zrow, None)
  for k in range(RPT // EC):
    pltpu.async_copy(rows0, acc.at[pl.ds(sid * RPT + k * EC, EC)], ssem[0])

  def _zdeg(i, _):
    dzero[pl.ds(i * 16, 16)] = jnp.zeros((16,), jnp.float32)
    return _
  lax.fori_loop(0, RPT // 16, _zdeg, None)
  pltpu.async_copy(dzero, dacc.at[pl.ds(sid * RPT, RPT)], ssem[1])

  for i in range(EC // 16):
    ones[pl.ds(i * 16, 16)] = jnp.ones((16,), jnp.float32)

  for k in range(RPT // EC):
    pltpu.make_async_copy(rows0, acc.at[pl.ds(sid * RPT, EC)], ssem[0]).wait()
  pltpu.make_async_copy(dzero, dacc.at[pl.ds(sid * RPT, RPT)], ssem[1]).wait()

  plsc.subcore_barrier()

  for k in range(3):
    pltpu.sync_copy(src_hbm.at[pl.ds(base + k * EC, EC)], sidx[k])
    pltpu.sync_copy(dst_hbm.at[pl.ds(base + k * EC, EC)], didx[k])
    pltpu.async_copy(x_hbm.at[sidx[k]], rowsb[k], gsem[k])
  pltpu.async_copy(src_hbm.at[pl.ds(base + 3 * EC, EC)], sidx[3], isem[3])

  def _step(i, b):
    p = (b + 3) % 4
    pltpu.make_async_copy(x_hbm.at[sidx[b]], rowsb[b], gsem[b]).wait()

    @pl.when(i >= 3)
    def _():
      pltpu.make_async_copy(dst_hbm.at[pl.ds(base, EC)], didx[b],
                            jsem[b]).wait()

    pltpu.async_copy(rowsb[b], acc.at[didx[b]], ssem[b], add=True)
    pltpu.async_copy(ones, dacc.at[didx[b]], dsem, add=True)

    @pl.when(i > 0)
    def _():
      pltpu.make_async_copy(rowsb[p], acc.at[didx[p]], ssem[p]).wait()
      pltpu.make_async_copy(ones, dacc.at[didx[p]], dsem).wait()

    @pl.when(i + 3 < NCH)
    def _():
      pltpu.make_async_copy(src_hbm.at[pl.ds(base, EC)], sidx[p],
                            isem[p]).wait()
      pltpu.async_copy(x_hbm.at[sidx[p]], rowsb[p], gsem[p])
      pltpu.async_copy(dst_hbm.at[pl.ds(base + (i + 3) * EC, EC)],
                       didx[p], jsem[p])

    @pl.when(i + 4 < NCH)
    def _():
      pltpu.async_copy(src_hbm.at[pl.ds(base + (i + 4) * EC, EC)],
                       sidx[b], isem[b])

  def _quad(g, _):
    for b in range(4):
      _step(4 * g + b, b)
    return _
  lax.fori_loop(0, NCH // 4, _quad, None)
  _step(NCH - 1, 0)

  pltpu.make_async_copy(rowsb[0], acc.at[didx[0]], ssem[0]).wait()
  pltpu.make_async_copy(ones, dacc.at[didx[0]], dsem).wait()

  plsc.subcore_barrier()

  sl = pl.ds(sid * RPT, RPT)
  pltpu.async_copy(acc.at[sl], psum_hbm.at[cid, sl], gsem[0])
  pltpu.sync_copy(dacc.at[sl], dzero)

  for k in range(RPT // EC):
    buf = rowsb[1 + (k % 2)]
    xsem = gsem[1 + (k % 2)]
    if k >= 2:
      pltpu.make_async_copy(
          buf, pdeg_hbm.at[cid, pl.ds(sid * RPT + (k - 2) * EC, EC)],
          xsem).wait()

    def _bg(g, _):
      dv = dzero[pl.ds(k * EC + g * 16, 16)]
      for j in range(16):
        s = jnp.full((16,), dv[j], jnp.float32)
        for c in range(D // 16):
          buf[g * 16 + j, pl.ds(c * 16, 16)] = s
      return _
    lax.fori_loop(0, EC // 16, _bg, None)
    pltpu.async_copy(buf, pdeg_hbm.at[cid, pl.ds(sid * RPT + k * EC, EC)],
                     xsem)

  for k in range(RPT // EC - 2, RPT // EC):
    pltpu.make_async_copy(
        rowsb[1 + (k % 2)],
        pdeg_hbm.at[cid, pl.ds(sid * RPT + k * EC, EC)],
        gsem[1 + (k % 2)]).wait()
  pltpu.make_async_copy(acc.at[sl], psum_hbm.at[cid, sl], gsem[0]).wait()


BN = 2000


def _mlp_body(x_ref, ps_ref, pd_ref, w1_ref, b1_ref, w2_ref, b2_ref, w3_ref,
              b3_ref, o_ref):
  deg = pd_ref[0] + pd_ref[1]
  has = deg > 0.0
  sn = jnp.where(has, 0.5 / jnp.maximum(deg, 1.0), 0.0)
  sx = jnp.where(has, 0.5, 1.0)
  a = x_ref[...] * sx + (ps_ref[0] + ps_ref[1]) * sn
  dn = (((1,), (1,)), ((), ()))
  bf = jnp.bfloat16
  h = lax.dot_general(a.astype(bf), w1_ref[...].astype(bf), dn,
                      preferred_element_type=jnp.float32)
  h = jnp.maximum(h + b1_ref[...], 0.0)
  h = lax.dot_general(h.astype(bf), w2_ref[...].astype(bf), dn,
                      preferred_element_type=jnp.float32)
  h = jnp.maximum(h + b2_ref[...], 0.0)
  o_ref[...] = jnp.sum(h * w3_ref[...], axis=1, keepdims=True) + b3_ref[...]


def _tc_mlp(x, psum, pdegx, W1, b1, W2, b2, W3, b3):
  return pl.pallas_call(
      _mlp_body,
      grid=(N // BN,),
      in_specs=[
          pl.BlockSpec((BN, D), lambda g: (g, 0)),
          pl.BlockSpec((NC, BN, D), lambda g: (0, g, 0)),
          pl.BlockSpec((NC, BN, D), lambda g: (0, g, 0)),
          pl.BlockSpec((H, D), lambda g: (0, 0)),
          pl.BlockSpec((1, H), lambda g: (0, 0)),
          pl.BlockSpec((H, H), lambda g: (0, 0)),
          pl.BlockSpec((1, H), lambda g: (0, 0)),
          pl.BlockSpec((1, H), lambda g: (0, 0)),
          pl.BlockSpec((1, 1), lambda g: (0, 0)),
      ],
      out_specs=pl.BlockSpec((BN, 1), lambda g: (g, 0)),
      out_shape=jax.ShapeDtypeStruct((N, 1), jnp.float32),
  )(x, psum, pdegx, W1, b1.reshape(1, H), W2, b2.reshape(1, H), W3,
    b3.reshape(1, 1))


def kernel(x, edge_index, W1, b1, W2, b2, W3, b3):
  src = edge_index[0]
  dst = edge_index[1]
  psum, pdegx = _sc_accumulate(x, src, dst)
  return _tc_mlp(x, psum, pdegx, W1, b1, W2, b2, W3, b3)
